# baseline (speedup 1.0000x reference)
import functools

import jax
import jax.numpy as jnp
from jax import lax
from jax.experimental import pallas as pl
from jax.experimental.pallas import tpu as pltpu
from jax.experimental.pallas import tpu_sc as plsc

N = 10000
NNZ = 320000
F = 128
NW = 32
CHUNK = 128
NCHUNK = 80
NNZ_PAD = NW * NCHUNK * CHUNK
DUMMY_GATHER = 0
DUMMY_SCATTER = N + 200
NPAD = 10240
ROWS_PER_TILE = NPAD // 16

_BLK = 1024
_GRID = NPAD // _BLK


def _mm_body(x_ref, act_ref, w1_ref, b1_ref, wup_ref, bup_ref, m_ref, xw_ref):
    x = x_ref[...]
    a = act_ref[...]
    send = a[:, 0:1] + a[:, 2:3]
    m = lax.dot_general(x * send, w1_ref[...], (((1,), (1,)), ((), ())),
                        preferred_element_type=jnp.float32)
    m_ref[...] = jnp.maximum(m + b1_ref[...], 0.0)
    xw = lax.dot_general(x, wup_ref[...], (((1,), (1,)), ((), ())),
                         preferred_element_type=jnp.float32)
    xw_ref[...] = xw + bup_ref[...]


def _tc_matmuls(x, action, W1, b1, W_up, b_up):
    return pl.pallas_call(
        _mm_body,
        grid=(10,),
        in_specs=[
            pl.BlockSpec((1000, F), lambda i: (i, 0)),
            pl.BlockSpec((1000, 3), lambda i: (i, 0)),
            pl.BlockSpec((F, F), lambda i: (0, 0)),
            pl.BlockSpec((1, F), lambda i: (0, 0)),
            pl.BlockSpec((F, F), lambda i: (0, 0)),
            pl.BlockSpec((1, F), lambda i: (0, 0)),
        ],
        out_specs=[
            pl.BlockSpec((1000, F), lambda i: (i, 0)),
            pl.BlockSpec((1000, F), lambda i: (i, 0)),
        ],
        out_shape=[
            jax.ShapeDtypeStruct((N, F), jnp.float32),
            jax.ShapeDtypeStruct((N, F), jnp.float32),
        ],
    )(x, action, W1, b1.reshape(1, F), W_up, b_up.reshape(1, F))


def _count_col(cnt_blk):
    c = cnt_blk.reshape(NW, cnt_blk.shape[-1])
    return lax.dot_general(c, jnp.ones((NW, 1), jnp.float32),
                           (((0,), (0,)), ((), ())),
                           preferred_element_type=jnp.float32)


def _mid_body(p_ref, cnt_ref, out_ref):
    s = p_ref[0] + p_ref[1]
    cnt = _count_col(cnt_ref[...])
    out_ref[...] = s / jnp.clip(cnt, 1.0, None)


def _tc_mid(p, cnt):
    return pl.pallas_call(
        _mid_body,
        grid=(_GRID,),
        in_specs=[pl.BlockSpec((2, _BLK, F), lambda i: (0, i, 0)),
                  pl.BlockSpec((NW, 1, _BLK), lambda i: (0, 0, i))],
        out_specs=pl.BlockSpec((_BLK, F), lambda i: (i, 0)),
        out_shape=jax.ShapeDtypeStruct((NPAD, F), jnp.float32),
    )(p, cnt)


def _fin_body(q_ref, cnt_ref, xw_ref, act_ref, out_ref):
    s = q_ref[0] + q_ref[1]
    cnt = _count_col(cnt_ref[...])
    m_i = s / jnp.clip(cnt, 1.0, None)
    a = act_ref[...]
    receive = a[:, 0:1] + a[:, 1:2]
    out_ref[...] = jnp.maximum(xw_ref[...] + m_i * receive, 0.0)


def _tc_final(q, cnt, xw, action):
    return pl.pallas_call(
        _fin_body,
        grid=(_GRID,),
        in_specs=[pl.BlockSpec((2, _BLK, F), lambda i: (0, i, 0)),
                  pl.BlockSpec((NW, 1, _BLK), lambda i: (0, 0, i)),
                  pl.BlockSpec((_BLK, F), lambda i: (i, 0)),
                  pl.BlockSpec((_BLK, 3), lambda i: (i, 0))],
        out_specs=pl.BlockSpec((_BLK, F), lambda i: (i, 0)),
        out_shape=jax.ShapeDtypeStruct((N, F), jnp.float32),
    )(q, cnt, xw, action)


def _sc_body(table_hbm, gidx_hbm, sidx_hbm, zeros_hbm, out_hbm, cnt_hbm,
             gidx_a, gidx_b, sidx_a, sidx_b, rows_a, rows_b, hist_v, acc_sh,
             sem_a, sem_b, sem_ia, sem_ib, sem_ja, sem_jb, sem_sa, sem_sb):
    cid = lax.axis_index("c")
    sid = lax.axis_index("s")
    wid = cid * 16 + sid

    pltpu.sync_copy(zeros_hbm.at[pl.ds(sid * ROWS_PER_TILE, ROWS_PER_TILE)],
                    acc_sh.at[pl.ds(sid * ROWS_PER_TILE, ROWS_PER_TILE)])

    def zero_body(i, carry):
        hist_v[pl.ds(i * 16, 16)] = jnp.zeros((16,), jnp.float32)
        return carry

    lax.fori_loop(0, NPAD // 16, zero_body, 0)
    plsc.subcore_barrier()

    cal, _ = plsc.scan_count(jnp.zeros((16,), jnp.int32))
    delta = 16 - lax.reduce_max(cal, (0,))

    def hist_chunk(sbuf):
        row = sbuf.at[0]
        for j0 in range(0, CHUNK, 16):
            idx16 = row[pl.ds(j0, 16)]
            rc, last = plsc.scan_count(idx16)
            plsc.addupdate_scatter(hist_v, [idx16],
                                   (rc + delta).astype(jnp.float32),
                                   mask=last)

    def idx_load(i, gidx, sem):
        pltpu.async_copy(gidx_hbm.at[wid].at[jnp.minimum(i, NCHUNK - 1)],
                         gidx, sem)

    def idx_wait(gidx, sem):
        pltpu.make_async_copy(gidx_hbm.at[wid].at[0], gidx, sem).wait()

    def sidx_load(i, sbuf, sem):
        pltpu.async_copy(
            sidx_hbm.at[wid].at[pl.ds(jnp.minimum(i, NCHUNK - 1), 1)],
            sbuf, sem)

    def sidx_wait(sbuf, sem):
        pltpu.make_async_copy(sidx_hbm.at[wid].at[pl.ds(0, 1)], sbuf,
                              sem).wait()

    def gather(gidx, rows, sem):
        pltpu.async_copy(table_hbm.at[gidx], rows, sem)

    def drain(rows, sem):
        pltpu.make_async_copy(zeros_hbm.at[pl.ds(0, CHUNK)], rows, sem).wait()

    def scatter(sbuf, rows, sem):
        pltpu.async_copy(rows, acc_sh.at[sbuf.at[0]], sem, add=True)

    idx_load(0, gidx_a, sem_ia)
    sidx_load(0, sidx_a, sem_ja)
    idx_wait(gidx_a, sem_ia)
    gather(gidx_a, rows_a, sem_a)
    idx_load(1, gidx_b, sem_ib)
    sidx_load(1, sidx_b, sem_jb)
    idx_wait(gidx_b, sem_ib)
    sidx_wait(sidx_a, sem_ja)

    def body(k, carry):
        i = 2 * k
        gather(gidx_b, rows_b, sem_b)
        drain(rows_a, sem_a)
        idx_load(i + 2, gidx_a, sem_ia)
        scatter(sidx_a, rows_a, sem_sa)
        hist_chunk(sidx_a)
        sidx_wait(sidx_b, sem_jb)
        idx_wait(gidx_a, sem_ia)
        drain(rows_a, sem_sa)
        sidx_load(i + 2, sidx_a, sem_ja)
        gather(gidx_a, rows_a, sem_a)
        drain(rows_b, sem_b)
        idx_load(i + 3, gidx_b, sem_ib)
        scatter(sidx_b, rows_b, sem_sb)
        hist_chunk(sidx_b)
        idx_wait(gidx_b, sem_ib)
        drain(rows_b, sem_sb)
        sidx_load(i + 3, sidx_b, sem_jb)
        sidx_wait(sidx_a, sem_ja)
        return carry

    lax.fori_loop(0, NCHUNK // 2 - 1, body, 0)
    gather(gidx_b, rows_b, sem_b)
    drain(rows_a, sem_a)
    scatter(sidx_a, rows_a, sem_sa)
    hist_chunk(sidx_a)
    sidx_wait(sidx_b, sem_jb)
    drain(rows_a, sem_sa)
    drain(rows_b, sem_b)
    scatter(sidx_b, rows_b, sem_sb)
    hist_chunk(sidx_b)
    drain(rows_b, sem_sb)
    plsc.subcore_barrier()
    pltpu.sync_copy(acc_sh.at[pl.ds(sid * ROWS_PER_TILE, ROWS_PER_TILE)],
                    out_hbm.at[cid].at[pl.ds(sid * ROWS_PER_TILE, ROWS_PER_TILE)])
    pltpu.sync_copy(hist_v, cnt_hbm.at[wid, 0])


@functools.cache
def _make_sc_agg(table_rows):
    return functools.partial(
        pl.kernel,
        mesh=plsc.VectorSubcoreMesh(core_axis_name="c", subcore_axis_name="s"),
        out_type=(
            jax.ShapeDtypeStruct((2, NPAD, F), jnp.float32),
            jax.ShapeDtypeStruct((NW, 1, NPAD), jnp.float32),
        ),
        compiler_params=pltpu.CompilerParams(needs_layout_passes=False),
        scratch_types=[
            pltpu.VMEM((CHUNK,), jnp.int32),
            pltpu.VMEM((CHUNK,), jnp.int32),
            pltpu.VMEM((1, CHUNK), jnp.int32),
            pltpu.VMEM((1, CHUNK), jnp.int32),
            pltpu.VMEM((CHUNK, F), jnp.float32),
            pltpu.VMEM((CHUNK, F), jnp.float32),
            pltpu.VMEM((NPAD,), jnp.float32),
            pltpu.VMEM_SHARED((NPAD, F), jnp.float32),
        ] + [pltpu.SemaphoreType.DMA] * 8,
    )(_sc_body)


def _pad_idx(idx, fill):
    pad = jnp.full((NNZ_PAD - NNZ,), fill, idx.dtype)
    return jnp.concatenate([idx, pad]).reshape(NW, NCHUNK, CHUNK)


def kernel(x, action, hyperedge_index, W1, b1, W_up, b_up):
    v_g = _pad_idx(hyperedge_index[0], DUMMY_GATHER)
    v_s = _pad_idx(hyperedge_index[0], DUMMY_SCATTER)
    e_g = _pad_idx(hyperedge_index[1], DUMMY_GATHER)
    e_s = _pad_idx(hyperedge_index[1], DUMMY_SCATTER)
    zeros = jnp.zeros((NPAD, F), jnp.float32)

    m, xw = _tc_matmuls(x, action, W1, b1, W_up, b_up)
    p, cnt_e = _make_sc_agg(N)(m, v_g, e_s, zeros)
    e_tab = _tc_mid(p, cnt_e)
    q, cnt_v = _make_sc_agg(NPAD)(e_tab, e_g, v_s, zeros)
    return _tc_final(q, cnt_v, xw, action)


# speedup vs baseline: 3.0784x; 3.0784x over previous
import functools

import jax
import jax.numpy as jnp
from jax import lax
from jax.experimental import pallas as pl
from jax.experimental.pallas import tpu as pltpu
from jax.experimental.pallas import tpu_sc as plsc

N = 10000
NNZ = 320000
F = 128
NW = 32
PAIRS_PER_W = NNZ // NW
CHUNK = 80
NCHUNK = PAIRS_PER_W // CHUNK
NPAD = 10240
ROWS_PER_TILE = NPAD // 16

_BLK = 1024
_GRID = NPAD // _BLK


def _mm_body(x_ref, act_ref, w1_ref, b1_ref, wup_ref, bup_ref, m_ref, xw_ref):
    x = x_ref[...]
    a = act_ref[...]
    send = a[:, 0:1] + a[:, 2:3]
    m = lax.dot_general(x * send, w1_ref[...], (((1,), (1,)), ((), ())),
                        preferred_element_type=jnp.float32)
    m_ref[...] = jnp.maximum(m + b1_ref[...], 0.0)
    xw = lax.dot_general(x, wup_ref[...], (((1,), (1,)), ((), ())),
                         preferred_element_type=jnp.float32)
    xw_ref[...] = xw + bup_ref[...]


def _tc_matmuls(x, action, W1, b1, W_up, b_up):
    return pl.pallas_call(
        _mm_body,
        grid=(10,),
        in_specs=[
            pl.BlockSpec((1000, F), lambda i: (i, 0)),
            pl.BlockSpec((1000, 3), lambda i: (i, 0)),
            pl.BlockSpec((F, F), lambda i: (0, 0)),
            pl.BlockSpec((1, F), lambda i: (0, 0)),
            pl.BlockSpec((F, F), lambda i: (0, 0)),
            pl.BlockSpec((1, F), lambda i: (0, 0)),
        ],
        out_specs=[
            pl.BlockSpec((1000, F), lambda i: (i, 0)),
            pl.BlockSpec((1000, F), lambda i: (i, 0)),
        ],
        out_shape=[
            jax.ShapeDtypeStruct((N, F), jnp.float32),
            jax.ShapeDtypeStruct((N, F), jnp.float32),
        ],
    )(x, action, W1, b1.reshape(1, F), W_up, b_up.reshape(1, F))


def _count_col(cnt_blk):
    c = cnt_blk.reshape(NW, cnt_blk.shape[-1])
    return lax.dot_general(c, jnp.ones((NW, 1), jnp.float32),
                           (((0,), (0,)), ((), ())),
                           preferred_element_type=jnp.float32)


def _mid_body(p_ref, cnt_ref, out_ref):
    s = p_ref[0] + p_ref[1]
    cnt = _count_col(cnt_ref[...])
    out_ref[...] = s / jnp.clip(cnt, 1.0, None)


def _tc_mid(p, cnt):
    return pl.pallas_call(
        _mid_body,
        grid=(_GRID,),
        in_specs=[pl.BlockSpec((2, _BLK, F), lambda i: (0, i, 0)),
                  pl.BlockSpec((NW, 1, _BLK), lambda i: (0, 0, i))],
        out_specs=pl.BlockSpec((_BLK, F), lambda i: (i, 0)),
        out_shape=jax.ShapeDtypeStruct((NPAD, F), jnp.float32),
    )(p, cnt)


def _fin_body(q_ref, cnt_ref, xw_ref, act_ref, out_ref):
    s = q_ref[0] + q_ref[1]
    cnt = _count_col(cnt_ref[...])
    m_i = s / jnp.clip(cnt, 1.0, None)
    a = act_ref[...]
    receive = a[:, 0:1] + a[:, 1:2]
    out_ref[...] = jnp.maximum(xw_ref[...] + m_i * receive, 0.0)


def _tc_final(q, cnt, xw, action):
    return pl.pallas_call(
        _fin_body,
        grid=(_GRID,),
        in_specs=[pl.BlockSpec((2, _BLK, F), lambda i: (0, i, 0)),
                  pl.BlockSpec((NW, 1, _BLK), lambda i: (0, 0, i)),
                  pl.BlockSpec((_BLK, F), lambda i: (i, 0)),
                  pl.BlockSpec((_BLK, 3), lambda i: (i, 0))],
        out_specs=pl.BlockSpec((_BLK, F), lambda i: (i, 0)),
        out_shape=jax.ShapeDtypeStruct((N, F), jnp.float32),
    )(q, cnt, xw, action)


def _sc_body(table_hbm, gidx_hbm, sidx_hbm, zeros_hbm, out_hbm, cnt_hbm,
             gidx_a, gidx_b, sidx_v, rows_a, rows_b, hist_v, acc_sh,
             sem_a, sem_b, sem_ia, sem_ib, sem_sa, sem_sb):
    cid = lax.axis_index("c")
    sid = lax.axis_index("s")
    wid = cid * 16 + sid

    cal, _ = plsc.scan_count(jnp.zeros((16,), jnp.int32))
    delta = 16 - lax.reduce_max(cal, (0,))

    def hist_chunk(i):
        row = sidx_v.at[i]
        for j0 in range(0, CHUNK, 16):
            idx16 = row[pl.ds(j0, 16)]
            rc, last = plsc.scan_count(idx16)
            plsc.addupdate_scatter(hist_v, [idx16],
                                   (rc + delta).astype(jnp.float32),
                                   mask=last)

    def idx_load(i, gidx, sem):
        pltpu.async_copy(gidx_hbm.at[wid].at[jnp.minimum(i, NCHUNK - 1)],
                         gidx, sem)

    def idx_wait(gidx, sem):
        pltpu.make_async_copy(gidx_hbm.at[wid].at[0], gidx, sem).wait()

    def gather(gidx, rows, sem):
        pltpu.async_copy(table_hbm.at[gidx], rows, sem)

    def drain(rows, sem):
        pltpu.make_async_copy(zeros_hbm.at[pl.ds(0, CHUNK)], rows, sem).wait()

    def scatter(i, rows, sem):
        pltpu.async_copy(rows, acc_sh.at[sidx_v.at[i]], sem, add=True)

    idx_load(0, gidx_a, sem_ia)
    idx_wait(gidx_a, sem_ia)
    gather(gidx_a, rows_a, sem_a)
    idx_load(1, gidx_b, sem_ib)

    pltpu.sync_copy(zeros_hbm.at[pl.ds(sid * ROWS_PER_TILE, ROWS_PER_TILE)],
                    acc_sh.at[pl.ds(sid * ROWS_PER_TILE, ROWS_PER_TILE)])
    pltpu.sync_copy(sidx_hbm.at[wid], sidx_v)

    def zero_body(i, carry):
        hist_v[pl.ds(i * 16, 16)] = jnp.zeros((16,), jnp.float32)
        return carry

    lax.fori_loop(0, NPAD // 16, zero_body, 0)
    idx_wait(gidx_b, sem_ib)
    plsc.subcore_barrier()

    def body(k, carry):
        i = 2 * k
        gather(gidx_b, rows_b, sem_b)
        drain(rows_a, sem_a)
        idx_load(i + 2, gidx_a, sem_ia)
        scatter(i, rows_a, sem_sa)
        hist_chunk(i)
        idx_wait(gidx_a, sem_ia)
        drain(rows_a, sem_sa)
        gather(gidx_a, rows_a, sem_a)
        drain(rows_b, sem_b)
        idx_load(i + 3, gidx_b, sem_ib)
        scatter(i + 1, rows_b, sem_sb)
        hist_chunk(i + 1)
        idx_wait(gidx_b, sem_ib)
        drain(rows_b, sem_sb)
        return carry

    lax.fori_loop(0, (NCHUNK - 1) // 2, body, 0)
    drain(rows_a, sem_a)
    scatter(NCHUNK - 1, rows_a, sem_sa)
    hist_chunk(NCHUNK - 1)
    drain(rows_a, sem_sa)
    plsc.subcore_barrier()
    pltpu.sync_copy(acc_sh.at[pl.ds(sid * ROWS_PER_TILE, ROWS_PER_TILE)],
                    out_hbm.at[cid].at[pl.ds(sid * ROWS_PER_TILE, ROWS_PER_TILE)])
    pltpu.sync_copy(hist_v, cnt_hbm.at[wid, 0])


@functools.cache
def _make_sc_agg(table_rows):
    return functools.partial(
        pl.kernel,
        mesh=plsc.VectorSubcoreMesh(core_axis_name="c", subcore_axis_name="s"),
        out_type=(
            jax.ShapeDtypeStruct((2, NPAD, F), jnp.float32),
            jax.ShapeDtypeStruct((NW, 1, NPAD), jnp.float32),
        ),
        compiler_params=pltpu.CompilerParams(needs_layout_passes=False),
        scratch_types=[
            pltpu.VMEM((CHUNK,), jnp.int32),
            pltpu.VMEM((CHUNK,), jnp.int32),
            pltpu.VMEM((NCHUNK, CHUNK), jnp.int32),
            pltpu.VMEM((CHUNK, F), jnp.float32),
            pltpu.VMEM((CHUNK, F), jnp.float32),
            pltpu.VMEM((NPAD,), jnp.float32),
            pltpu.VMEM_SHARED((NPAD, F), jnp.float32),
            pltpu.SemaphoreType.DMA,
            pltpu.SemaphoreType.DMA,
            pltpu.SemaphoreType.DMA,
            pltpu.SemaphoreType.DMA,
            pltpu.SemaphoreType.DMA,
            pltpu.SemaphoreType.DMA,
        ],
    )(_sc_body)


def kernel(x, action, hyperedge_index, W1, b1, W_up, b_up):
    v_idx = hyperedge_index[0].reshape(NW, NCHUNK, CHUNK)
    e_idx = hyperedge_index[1].reshape(NW, NCHUNK, CHUNK)
    zeros = jnp.zeros((NPAD, F), jnp.float32)

    m, xw = _tc_matmuls(x, action, W1, b1, W_up, b_up)
    p, cnt_e = _make_sc_agg(N)(m, v_idx, e_idx, zeros)
    e_tab = _tc_mid(p, cnt_e)
    q, cnt_v = _make_sc_agg(NPAD)(e_tab, e_idx, v_idx, zeros)
    return _tc_final(q, cnt_v, xw, action)


# speedup vs baseline: 3.0834x; 1.0016x over previous
import functools

import jax
import jax.numpy as jnp
from jax import lax
from jax.experimental import pallas as pl
from jax.experimental.pallas import tpu as pltpu
from jax.experimental.pallas import tpu_sc as plsc

N = 10000
NNZ = 320000
F = 128
NW = 32
PAIRS_PER_W = NNZ // NW
CHUNK = 80
NCHUNK = PAIRS_PER_W // CHUNK
NPAD = 10240
ROWS_PER_TILE = NPAD // 16

_BLK = 1024
_GRID = NPAD // _BLK


def _mm_body(x_ref, act_ref, w1_ref, b1_ref, wup_ref, bup_ref, m_ref, xw_ref):
    x = x_ref[...]
    a = act_ref[...]
    send = a[:, 0:1] + a[:, 2:3]
    m = lax.dot_general(x * send, w1_ref[...], (((1,), (1,)), ((), ())),
                        preferred_element_type=jnp.float32)
    m_ref[...] = jnp.maximum(m + b1_ref[...], 0.0)
    xw = lax.dot_general(x, wup_ref[...], (((1,), (1,)), ((), ())),
                         preferred_element_type=jnp.float32)
    xw_ref[...] = xw + bup_ref[...]


def _tc_matmuls(x, action, W1, b1, W_up, b_up):
    return pl.pallas_call(
        _mm_body,
        grid=(10,),
        in_specs=[
            pl.BlockSpec((1000, F), lambda i: (i, 0)),
            pl.BlockSpec((1000, 3), lambda i: (i, 0)),
            pl.BlockSpec((F, F), lambda i: (0, 0)),
            pl.BlockSpec((1, F), lambda i: (0, 0)),
            pl.BlockSpec((F, F), lambda i: (0, 0)),
            pl.BlockSpec((1, F), lambda i: (0, 0)),
        ],
        out_specs=[
            pl.BlockSpec((1000, F), lambda i: (i, 0)),
            pl.BlockSpec((1000, F), lambda i: (i, 0)),
        ],
        out_shape=[
            jax.ShapeDtypeStruct((N, F), jnp.float32),
            jax.ShapeDtypeStruct((N, F), jnp.float32),
        ],
    )(x, action, W1, b1.reshape(1, F), W_up, b_up.reshape(1, F))


def _count_col(cnt_blk):
    c = cnt_blk.reshape(NW, cnt_blk.shape[-1])
    return lax.dot_general(c, jnp.ones((NW, 1), jnp.float32),
                           (((0,), (0,)), ((), ())),
                           preferred_element_type=jnp.float32)


def _mid_body(p_ref, cnt_ref, out_ref):
    s = p_ref[0] + p_ref[1]
    cnt = _count_col(cnt_ref[...])
    out_ref[...] = s / jnp.clip(cnt, 1.0, None)


def _tc_mid(p, cnt):
    return pl.pallas_call(
        _mid_body,
        grid=(_GRID,),
        in_specs=[pl.BlockSpec((2, _BLK, F), lambda i: (0, i, 0)),
                  pl.BlockSpec((NW, 1, _BLK), lambda i: (0, 0, i))],
        out_specs=pl.BlockSpec((_BLK, F), lambda i: (i, 0)),
        out_shape=jax.ShapeDtypeStruct((NPAD, F), jnp.float32),
    )(p, cnt)


def _fin_body(q_ref, cnt_ref, xw_ref, act_ref, out_ref):
    s = q_ref[0] + q_ref[1]
    cnt = _count_col(cnt_ref[...])
    m_i = s / jnp.clip(cnt, 1.0, None)
    a = act_ref[...]
    receive = a[:, 0:1] + a[:, 1:2]
    out_ref[...] = jnp.maximum(xw_ref[...] + m_i * receive, 0.0)


def _tc_final(q, cnt, xw, action):
    return pl.pallas_call(
        _fin_body,
        grid=(_GRID,),
        in_specs=[pl.BlockSpec((2, _BLK, F), lambda i: (0, i, 0)),
                  pl.BlockSpec((NW, 1, _BLK), lambda i: (0, 0, i)),
                  pl.BlockSpec((_BLK, F), lambda i: (i, 0)),
                  pl.BlockSpec((_BLK, 3), lambda i: (i, 0))],
        out_specs=pl.BlockSpec((_BLK, F), lambda i: (i, 0)),
        out_shape=jax.ShapeDtypeStruct((N, F), jnp.float32),
    )(q, cnt, xw, action)


def _sc_body(table_hbm, gidx_hbm, sidx_hbm, zeros_hbm, out_hbm, cnt_hbm,
             gidx_a, gidx_b, sidx_v, rows_a, rows_b, hist_v, acc_sh,
             sem_a, sem_b, sem_ia, sem_ib, sem_sa, sem_sb):
    cid = lax.axis_index("c")
    sid = lax.axis_index("s")
    wid = cid * 16 + sid

    cal, _ = plsc.scan_count(jnp.zeros((16,), jnp.int32))
    delta = 16 - lax.reduce_max(cal, (0,))

    def hist_chunk(i):
        row = sidx_v.at[i]
        for j0 in range(0, CHUNK, 16):
            idx16 = row[pl.ds(j0, 16)]
            rc, last = plsc.scan_count(idx16)
            plsc.addupdate_scatter(hist_v, [idx16],
                                   (rc + delta).astype(jnp.float32),
                                   mask=last)

    def idx_load(i, gidx, sem):
        pltpu.async_copy(gidx_hbm.at[wid].at[jnp.minimum(i, NCHUNK - 1)],
                         gidx, sem)

    def idx_wait(gidx, sem):
        pltpu.make_async_copy(gidx_hbm.at[wid].at[0], gidx, sem).wait()

    def gather(gidx, rows, sem):
        h = CHUNK // 2
        pltpu.async_copy(table_hbm.at[gidx.at[pl.ds(0, h)]],
                         rows.at[pl.ds(0, h)], sem)
        pltpu.async_copy(table_hbm.at[gidx.at[pl.ds(h, h)]],
                         rows.at[pl.ds(h, h)], sem)

    def drain(rows, sem):
        pltpu.make_async_copy(zeros_hbm.at[pl.ds(0, CHUNK)], rows, sem).wait()

    def scatter(i, rows, sem):
        pltpu.async_copy(rows, acc_sh.at[sidx_v.at[i]], sem, add=True)

    idx_load(0, gidx_a, sem_ia)
    idx_wait(gidx_a, sem_ia)
    gather(gidx_a, rows_a, sem_a)
    idx_load(1, gidx_b, sem_ib)

    pltpu.sync_copy(zeros_hbm.at[pl.ds(sid * ROWS_PER_TILE, ROWS_PER_TILE)],
                    acc_sh.at[pl.ds(sid * ROWS_PER_TILE, ROWS_PER_TILE)])
    pltpu.sync_copy(sidx_hbm.at[wid], sidx_v)

    def zero_body(i, carry):
        hist_v[pl.ds(i * 16, 16)] = jnp.zeros((16,), jnp.float32)
        return carry

    lax.fori_loop(0, NPAD // 16, zero_body, 0)
    idx_wait(gidx_b, sem_ib)
    plsc.subcore_barrier()

    def body(k, carry):
        i = 2 * k
        gather(gidx_b, rows_b, sem_b)
        drain(rows_a, sem_a)
        idx_load(i + 2, gidx_a, sem_ia)
        scatter(i, rows_a, sem_sa)
        hist_chunk(i)
        idx_wait(gidx_a, sem_ia)
        drain(rows_a, sem_sa)
        gather(gidx_a, rows_a, sem_a)
        drain(rows_b, sem_b)
        idx_load(i + 3, gidx_b, sem_ib)
        scatter(i + 1, rows_b, sem_sb)
        hist_chunk(i + 1)
        idx_wait(gidx_b, sem_ib)
        drain(rows_b, sem_sb)
        return carry

    lax.fori_loop(0, (NCHUNK - 1) // 2, body, 0)
    drain(rows_a, sem_a)
    scatter(NCHUNK - 1, rows_a, sem_sa)
    hist_chunk(NCHUNK - 1)
    drain(rows_a, sem_sa)
    plsc.subcore_barrier()
    pltpu.sync_copy(acc_sh.at[pl.ds(sid * ROWS_PER_TILE, ROWS_PER_TILE)],
                    out_hbm.at[cid].at[pl.ds(sid * ROWS_PER_TILE, ROWS_PER_TILE)])
    pltpu.sync_copy(hist_v, cnt_hbm.at[wid, 0])


@functools.cache
def _make_sc_agg(table_rows):
    return functools.partial(
        pl.kernel,
        mesh=plsc.VectorSubcoreMesh(core_axis_name="c", subcore_axis_name="s"),
        out_type=(
            jax.ShapeDtypeStruct((2, NPAD, F), jnp.float32),
            jax.ShapeDtypeStruct((NW, 1, NPAD), jnp.float32),
        ),
        compiler_params=pltpu.CompilerParams(needs_layout_passes=False),
        scratch_types=[
            pltpu.VMEM((CHUNK,), jnp.int32),
            pltpu.VMEM((CHUNK,), jnp.int32),
            pltpu.VMEM((NCHUNK, CHUNK), jnp.int32),
            pltpu.VMEM((CHUNK, F), jnp.float32),
            pltpu.VMEM((CHUNK, F), jnp.float32),
            pltpu.VMEM((NPAD,), jnp.float32),
            pltpu.VMEM_SHARED((NPAD, F), jnp.float32),
            pltpu.SemaphoreType.DMA,
            pltpu.SemaphoreType.DMA,
            pltpu.SemaphoreType.DMA,
            pltpu.SemaphoreType.DMA,
            pltpu.SemaphoreType.DMA,
            pltpu.SemaphoreType.DMA,
        ],
    )(_sc_body)


def kernel(x, action, hyperedge_index, W1, b1, W_up, b_up):
    v_idx = hyperedge_index[0].reshape(NW, NCHUNK, CHUNK)
    e_idx = hyperedge_index[1].reshape(NW, NCHUNK, CHUNK)
    zeros = jnp.zeros((NPAD, F), jnp.float32)

    m, xw = _tc_matmuls(x, action, W1, b1, W_up, b_up)
    p, cnt_e = _make_sc_agg(N)(m, v_idx, e_idx, zeros)
    e_tab = _tc_mid(p, cnt_e)
    q, cnt_v = _make_sc_agg(NPAD)(e_tab, e_idx, v_idx, zeros)
    return _tc_final(q, cnt_v, xw, action)


# speedup vs baseline: 3.1017x; 1.0059x over previous
import functools

import jax
import jax.numpy as jnp
from jax import lax
from jax.experimental import pallas as pl
from jax.experimental.pallas import tpu as pltpu
from jax.experimental.pallas import tpu_sc as plsc

N = 10000
NNZ = 320000
F = 128
NW = 32
PAIRS_PER_W = NNZ // NW
CHUNK = 80
NCHUNK = PAIRS_PER_W // CHUNK
NPAD = 10240
ROWS_PER_TILE = NPAD // 16

_BLK = 1024
_GRID = NPAD // _BLK


def _mm_body(x_ref, act_ref, w1_ref, b1_ref, m_ref):
    x = x_ref[...]
    a = act_ref[...]
    send = a[:, 0:1] + a[:, 2:3]
    m = lax.dot_general(x * send, w1_ref[...], (((1,), (1,)), ((), ())),
                        preferred_element_type=jnp.float32)
    m_ref[...] = jnp.maximum(m + b1_ref[...], 0.0)


def _tc_matmuls(x, action, W1, b1):
    return pl.pallas_call(
        _mm_body,
        grid=(10,),
        in_specs=[
            pl.BlockSpec((1000, F), lambda i: (i, 0)),
            pl.BlockSpec((1000, 3), lambda i: (i, 0)),
            pl.BlockSpec((F, F), lambda i: (0, 0)),
            pl.BlockSpec((1, F), lambda i: (0, 0)),
        ],
        out_specs=pl.BlockSpec((1000, F), lambda i: (i, 0)),
        out_shape=jax.ShapeDtypeStruct((N, F), jnp.float32),
    )(x, action, W1, b1.reshape(1, F))


def _count_col(cnt_blk):
    c = cnt_blk.reshape(NW, cnt_blk.shape[-1])
    return lax.dot_general(c, jnp.ones((NW, 1), jnp.float32),
                           (((0,), (0,)), ((), ())),
                           preferred_element_type=jnp.float32)


def _mid_body(p_ref, cnt_ref, out_ref):
    s = p_ref[0] + p_ref[1]
    cnt = _count_col(cnt_ref[...])
    out_ref[...] = s / jnp.clip(cnt, 1.0, None)


def _tc_mid(p, cnt):
    return pl.pallas_call(
        _mid_body,
        grid=(_GRID,),
        in_specs=[pl.BlockSpec((2, _BLK, F), lambda i: (0, i, 0)),
                  pl.BlockSpec((NW, 1, _BLK), lambda i: (0, 0, i))],
        out_specs=pl.BlockSpec((_BLK, F), lambda i: (i, 0)),
        out_shape=jax.ShapeDtypeStruct((NPAD, F), jnp.float32),
    )(p, cnt)


def _fin_body(q_ref, cnt_ref, x_ref, act_ref, wup_ref, bup_ref, out_ref):
    s = q_ref[0] + q_ref[1]
    cnt = _count_col(cnt_ref[...])
    m_i = s / jnp.clip(cnt, 1.0, None)
    a = act_ref[...]
    receive = a[:, 0:1] + a[:, 1:2]
    xw = lax.dot_general(x_ref[...], wup_ref[...], (((1,), (1,)), ((), ())),
                         preferred_element_type=jnp.float32)
    out_ref[...] = jnp.maximum(xw + bup_ref[...] + m_i * receive, 0.0)


def _tc_final(q, cnt, x, action, W_up, b_up):
    return pl.pallas_call(
        _fin_body,
        grid=(_GRID,),
        in_specs=[pl.BlockSpec((2, _BLK, F), lambda i: (0, i, 0)),
                  pl.BlockSpec((NW, 1, _BLK), lambda i: (0, 0, i)),
                  pl.BlockSpec((_BLK, F), lambda i: (i, 0)),
                  pl.BlockSpec((_BLK, 3), lambda i: (i, 0)),
                  pl.BlockSpec((F, F), lambda i: (0, 0)),
                  pl.BlockSpec((1, F), lambda i: (0, 0))],
        out_specs=pl.BlockSpec((_BLK, F), lambda i: (i, 0)),
        out_shape=jax.ShapeDtypeStruct((N, F), jnp.float32),
    )(q, cnt, x, action, W_up, b_up.reshape(1, F))


def _sc_body(table_hbm, gidx_hbm, sidx_hbm, zeros_hbm, out_hbm, cnt_hbm,
             gidx_a, gidx_b, sidx_v, rows_a, rows_b, hist_v, acc_sh,
             sem_a, sem_b, sem_ia, sem_ib, sem_sa, sem_sb):
    cid = lax.axis_index("c")
    sid = lax.axis_index("s")
    wid = cid * 16 + sid

    cal, _ = plsc.scan_count(jnp.zeros((16,), jnp.int32))
    delta = 16 - lax.reduce_max(cal, (0,))

    def hist_chunk(i):
        row = sidx_v.at[i]
        for j0 in range(0, CHUNK, 16):
            idx16 = row[pl.ds(j0, 16)]
            rc, last = plsc.scan_count(idx16)
            plsc.addupdate_scatter(hist_v, [idx16],
                                   (rc + delta).astype(jnp.float32),
                                   mask=last)

    def idx_load(i, gidx, sem):
        pltpu.async_copy(gidx_hbm.at[wid].at[jnp.minimum(i, NCHUNK - 1)],
                         gidx, sem)

    def idx_wait(gidx, sem):
        pltpu.make_async_copy(gidx_hbm.at[wid].at[0], gidx, sem).wait()

    def gather(gidx, rows, sem):
        h = CHUNK // 2
        pltpu.async_copy(table_hbm.at[gidx.at[pl.ds(0, h)]],
                         rows.at[pl.ds(0, h)], sem)
        pltpu.async_copy(table_hbm.at[gidx.at[pl.ds(h, h)]],
                         rows.at[pl.ds(h, h)], sem)

    def drain(rows, sem):
        pltpu.make_async_copy(zeros_hbm.at[pl.ds(0, CHUNK)], rows, sem).wait()

    def scatter(i, rows, sem):
        pltpu.async_copy(rows, acc_sh.at[sidx_v.at[i]], sem, add=True)

    idx_load(0, gidx_a, sem_ia)
    idx_wait(gidx_a, sem_ia)
    gather(gidx_a, rows_a, sem_a)
    idx_load(1, gidx_b, sem_ib)

    pltpu.sync_copy(zeros_hbm.at[pl.ds(sid * ROWS_PER_TILE, ROWS_PER_TILE)],
                    acc_sh.at[pl.ds(sid * ROWS_PER_TILE, ROWS_PER_TILE)])
    pltpu.sync_copy(sidx_hbm.at[wid], sidx_v)

    def zero_body(i, carry):
        hist_v[pl.ds(i * 16, 16)] = jnp.zeros((16,), jnp.float32)
        return carry

    lax.fori_loop(0, NPAD // 16, zero_body, 0)
    idx_wait(gidx_b, sem_ib)
    plsc.subcore_barrier()

    def body(k, carry):
        i = 2 * k
        gather(gidx_b, rows_b, sem_b)
        drain(rows_a, sem_a)
        idx_load(i + 2, gidx_a, sem_ia)
        scatter(i, rows_a, sem_sa)
        hist_chunk(i)
        idx_wait(gidx_a, sem_ia)
        drain(rows_a, sem_sa)
        gather(gidx_a, rows_a, sem_a)
        drain(rows_b, sem_b)
        idx_load(i + 3, gidx_b, sem_ib)
        scatter(i + 1, rows_b, sem_sb)
        hist_chunk(i + 1)
        idx_wait(gidx_b, sem_ib)
        drain(rows_b, sem_sb)
        return carry

    lax.fori_loop(0, (NCHUNK - 1) // 2, body, 0)
    drain(rows_a, sem_a)
    scatter(NCHUNK - 1, rows_a, sem_sa)
    hist_chunk(NCHUNK - 1)
    drain(rows_a, sem_sa)
    plsc.subcore_barrier()
    pltpu.sync_copy(acc_sh.at[pl.ds(sid * ROWS_PER_TILE, ROWS_PER_TILE)],
                    out_hbm.at[cid].at[pl.ds(sid * ROWS_PER_TILE, ROWS_PER_TILE)])
    pltpu.sync_copy(hist_v, cnt_hbm.at[wid, 0])


@functools.cache
def _make_sc_agg(table_rows):
    return functools.partial(
        pl.kernel,
        mesh=plsc.VectorSubcoreMesh(core_axis_name="c", subcore_axis_name="s"),
        out_type=(
            jax.ShapeDtypeStruct((2, NPAD, F), jnp.float32),
            jax.ShapeDtypeStruct((NW, 1, NPAD), jnp.float32),
        ),
        compiler_params=pltpu.CompilerParams(needs_layout_passes=False),
        scratch_types=[
            pltpu.VMEM((CHUNK,), jnp.int32),
            pltpu.VMEM((CHUNK,), jnp.int32),
            pltpu.VMEM((NCHUNK, CHUNK), jnp.int32),
            pltpu.VMEM((CHUNK, F), jnp.float32),
            pltpu.VMEM((CHUNK, F), jnp.float32),
            pltpu.VMEM((NPAD,), jnp.float32),
            pltpu.VMEM_SHARED((NPAD, F), jnp.float32),
            pltpu.SemaphoreType.DMA,
            pltpu.SemaphoreType.DMA,
            pltpu.SemaphoreType.DMA,
            pltpu.SemaphoreType.DMA,
            pltpu.SemaphoreType.DMA,
            pltpu.SemaphoreType.DMA,
        ],
    )(_sc_body)


def kernel(x, action, hyperedge_index, W1, b1, W_up, b_up):
    v_idx = hyperedge_index[0].reshape(NW, NCHUNK, CHUNK)
    e_idx = hyperedge_index[1].reshape(NW, NCHUNK, CHUNK)
    zeros = jnp.zeros((NPAD, F), jnp.float32)

    m = _tc_matmuls(x, action, W1, b1)
    p, cnt_e = _make_sc_agg(N)(m, v_idx, e_idx, zeros)
    e_tab = _tc_mid(p, cnt_e)
    q, cnt_v = _make_sc_agg(NPAD)(e_tab, e_idx, v_idx, zeros)
    return _tc_final(q, cnt_v, x, action, W_up, b_up)
